# baseline (speedup 1.0000x reference)
import functools

import jax
import jax.numpy as jnp
from jax import lax
from jax.experimental import pallas as pl
from jax.experimental.pallas import tpu as pltpu
from jax.experimental.pallas import tpu_sc as plsc

_B, _C, _E, _P = 32, 32, 128, 256
_L = 16
_ROWS = 64
_NBUF = 3
_CHUNKS_PER_C = _E // _ROWS
_UNITS = _C * _CHUNKS_PER_C
_NGROUPS = _UNITS // _NBUF
_NTAIL = _UNITS - _NGROUPS * _NBUF


def _ppt_sc(X, perm_idx):
    mesh = plsc.VectorSubcoreMesh(core_axis_name="c", subcore_axis_name="s")

    @functools.partial(
        pl.kernel,
        out_type=jax.ShapeDtypeStruct((_B, _C, _E, _P), jnp.float32),
        mesh=mesh,
        compiler_params=pltpu.CompilerParams(
            needs_layout_passes=False, use_tc_tiling_on_sc=True),
        scratch_types=[
            pltpu.VMEM((_C, _P), jnp.int32),
            [pltpu.VMEM((_ROWS, _P), jnp.float32) for _ in range(_NBUF)],
            [pltpu.VMEM((_ROWS, _P), jnp.float32) for _ in range(_NBUF)],
            [pltpu.SemaphoreType.DMA for _ in range(_NBUF)],
            [pltpu.SemaphoreType.DMA for _ in range(_NBUF)],
        ],
    )
    def k(x_hbm, perm_hbm, out_hbm, perm_v, ibufs, obufs, isems, osems):
        w = lax.axis_index("s") * 2 + lax.axis_index("c")
        pltpu.sync_copy(perm_hbm, perm_v)

        def start_in(u, kbuf):
            c, r0 = u // _CHUNKS_PER_C, (u % _CHUNKS_PER_C) * _ROWS
            pltpu.async_copy(
                x_hbm.at[w, c, pl.ds(r0, _ROWS)], ibufs[kbuf], isems[kbuf])

        def wait_in(kbuf):
            pltpu.make_async_copy(
                x_hbm.at[0, 0, pl.ds(0, _ROWS)], ibufs[kbuf], isems[kbuf]).wait()

        def start_out(u, kbuf):
            c, r0 = u // _CHUNKS_PER_C, (u % _CHUNKS_PER_C) * _ROWS
            pltpu.async_copy(
                obufs[kbuf], out_hbm.at[w, c, pl.ds(r0, _ROWS)], osems[kbuf])

        def wait_out(kbuf):
            pltpu.make_async_copy(
                obufs[kbuf], out_hbm.at[0, 0, pl.ds(0, _ROWS)], osems[kbuf]).wait()

        def start_out_half(u, kbuf, h):
            c, r0 = u // _CHUNKS_PER_C, (u % _CHUNKS_PER_C) * _ROWS
            hr = _ROWS // 2
            pltpu.async_copy(
                obufs[kbuf].at[pl.ds(h * hr, hr)],
                out_hbm.at[w, c, pl.ds(r0 + h * hr, hr)], osems[kbuf])

        def permute_chunk(u, kbuf):
            c = u // _CHUNKS_PER_C
            ibuf, obuf = ibufs[kbuf], obufs[kbuf]
            hr = _ROWS // 2

            for h in range(2):
                def do_j(j, carry, h=h):
                    cols = perm_v[c, pl.ds(j * _L, _L)]
                    rows0 = jnp.full((_L,), h * hr, jnp.int32)

                    @plsc.parallel_loop(h * hr, (h + 1) * hr, unroll=8,
                                        carry=rows0)
                    def _(r, rows):
                        obuf[r, pl.ds(j * _L, _L)] = plsc.load_gather(
                            ibuf, [rows, cols])
                        return rows + 1

                    return carry

                lax.fori_loop(0, _P // _L, do_j, 0, unroll=False)
                start_out_half(u, kbuf, h)

        for kbuf in range(_NBUF):
            start_in(kbuf, kbuf)

        def do_group(g, carry):
            for kbuf in range(_NBUF):
                u = g * _NBUF + kbuf
                wait_in(kbuf)

                @pl.when(g > 0)
                def _():
                    wait_out(kbuf)

                permute_chunk(u, kbuf)

                @pl.when(u + _NBUF < _UNITS)
                def _():
                    start_in(u + _NBUF, kbuf)

            return carry

        lax.fori_loop(0, _NGROUPS, do_group, 0, unroll=False)

        for t in range(_NTAIL):
            u = _NGROUPS * _NBUF + t
            wait_in(t)
            wait_out(t)
            permute_chunk(u, t)
        for kbuf in range(_NBUF):
            wait_out(kbuf)

    return k(X, perm_idx)


def kernel(X, perm_idx):
    return _ppt_sc(X, perm_idx)


# speedup vs baseline: 1.0660x; 1.0660x over previous
import functools

import jax
import jax.numpy as jnp
from jax import lax
from jax.experimental import pallas as pl
from jax.experimental.pallas import tpu as pltpu
from jax.experimental.pallas import tpu_sc as plsc

_B, _C, _E, _P = 32, 32, 128, 256
_L = 16
_ROWS = 64
_NBUF = 3
_CHUNKS_PER_C = _E // _ROWS
_UNITS = _C * _CHUNKS_PER_C
_NGROUPS = _UNITS // _NBUF
_NTAIL = _UNITS - _NGROUPS * _NBUF


def _ppt_sc(X, perm_idx):
    mesh = plsc.VectorSubcoreMesh(core_axis_name="c", subcore_axis_name="s")

    @functools.partial(
        pl.kernel,
        out_type=jax.ShapeDtypeStruct((_B, _C, _E, _P), jnp.float32),
        mesh=mesh,
        compiler_params=pltpu.CompilerParams(
            needs_layout_passes=False, use_tc_tiling_on_sc=True),
        scratch_types=[
            pltpu.VMEM((_C, _P), jnp.int32),
            [pltpu.VMEM((_ROWS, _P), jnp.float32) for _ in range(_NBUF)],
            [pltpu.VMEM((_ROWS, _P), jnp.float32) for _ in range(_NBUF)],
            [pltpu.SemaphoreType.DMA for _ in range(_NBUF)],
            [pltpu.SemaphoreType.DMA for _ in range(_NBUF)],
        ],
    )
    def k(x_hbm, perm_hbm, out_hbm, perm_v, ibufs, obufs, isems, osems):
        w = lax.axis_index("s") * 2 + lax.axis_index("c")
        pltpu.sync_copy(perm_hbm, perm_v)

        def start_in(u, kbuf):
            c, r0 = u // _CHUNKS_PER_C, (u % _CHUNKS_PER_C) * _ROWS
            pltpu.async_copy(
                x_hbm.at[w, c, pl.ds(r0, _ROWS)], ibufs[kbuf], isems[kbuf])

        def wait_in(kbuf):
            pltpu.make_async_copy(
                x_hbm.at[0, 0, pl.ds(0, _ROWS)], ibufs[kbuf], isems[kbuf]).wait()

        def start_out(u, kbuf):
            c, r0 = u // _CHUNKS_PER_C, (u % _CHUNKS_PER_C) * _ROWS
            pltpu.async_copy(
                obufs[kbuf], out_hbm.at[w, c, pl.ds(r0, _ROWS)], osems[kbuf])

        def wait_out(kbuf):
            pltpu.make_async_copy(
                obufs[kbuf], out_hbm.at[0, 0, pl.ds(0, _ROWS)], osems[kbuf]).wait()

        def start_out_half(u, kbuf, h):
            c, r0 = u // _CHUNKS_PER_C, (u % _CHUNKS_PER_C) * _ROWS
            hr = _ROWS // 2
            pltpu.async_copy(
                obufs[kbuf].at[pl.ds(h * hr, hr)],
                out_hbm.at[w, c, pl.ds(r0 + h * hr, hr)], osems[kbuf])

        def permute_chunk(u, kbuf):
            c = u // _CHUNKS_PER_C
            ibuf, obuf = ibufs[kbuf], obufs[kbuf]

            @plsc.parallel_loop(0, _P // _L)
            def _(j):
                cols = perm_v[c, pl.ds(j * _L, _L)]
                rows0 = jnp.zeros((_L,), jnp.int32)

                @plsc.parallel_loop(0, _ROWS, unroll=8, carry=rows0)
                def _(r, rows):
                    obuf[r, pl.ds(j * _L, _L)] = plsc.load_gather(
                        ibuf, [rows, cols])
                    return rows + 1

        for kbuf in range(_NBUF):
            start_in(kbuf, kbuf)

        def do_group(g, carry):
            for kbuf in range(_NBUF):
                u = g * _NBUF + kbuf
                wait_in(kbuf)

                @pl.when(g > 0)
                def _():
                    wait_out(kbuf)

                permute_chunk(u, kbuf)
                start_out(u, kbuf)

                @pl.when(u + _NBUF < _UNITS)
                def _():
                    start_in(u + _NBUF, kbuf)

            return carry

        lax.fori_loop(0, _NGROUPS, do_group, 0, unroll=False)

        for t in range(_NTAIL):
            u = _NGROUPS * _NBUF + t
            wait_in(t)
            wait_out(t)
            permute_chunk(u, t)
            start_out(u, t)
        for kbuf in range(_NBUF):
            wait_out(kbuf)

    return k(X, perm_idx)


def kernel(X, perm_idx):
    return _ppt_sc(X, perm_idx)
